# baseline (speedup 1.0000x reference)
import functools

import jax
import jax.numpy as jnp
from jax import lax
from jax.experimental import pallas as pl
from jax.experimental.pallas import tpu as pltpu
from jax.experimental.pallas import tpu_sc as plsc

BATCH = 16384
NUM_WORKERS = 32
CHUNK = BATCH // NUM_WORKERS
LANES = 16
STEPS = CHUNK // LANES
UNROLL = 4


@functools.partial(
    pl.kernel,
    mesh=plsc.VectorSubcoreMesh(core_axis_name="c", subcore_axis_name="s"),
    out_type=jax.ShapeDtypeStruct((BATCH,), jnp.float32),
    scratch_types=[
        pltpu.VMEM((CHUNK,), jnp.int32),
        pltpu.VMEM((CHUNK,), jnp.float32),
        pltpu.VMEM((CHUNK,), jnp.float32),
        pltpu.VMEM((LANES,), jnp.float32),
        pltpu.SemaphoreType.DMA,
        pltpu.SemaphoreType.DMA,
        pltpu.SemaphoreType.DMA,
    ],
)
def _sc_forward(m1_hbm, m2_hbm, par_hbm, out_hbm,
                m1_v, m2_v, out_v, par_v, sem_a, sem_b, sem_o):
    wid = lax.axis_index("s") * 2 + lax.axis_index("c")
    base = wid * CHUNK
    HALF = CHUNK // 2
    in_a = [
        pltpu.async_copy(m1_hbm.at[pl.ds(base, HALF)],
                         m1_v.at[pl.ds(0, HALF)], sem_a),
        pltpu.async_copy(m2_hbm.at[pl.ds(base, HALF)],
                         m2_v.at[pl.ds(0, HALF)], sem_a),
        pltpu.async_copy(par_hbm, par_v, sem_a),
    ]
    in_b = [
        pltpu.async_copy(m1_hbm.at[pl.ds(base + HALF, HALF)],
                         m1_v.at[pl.ds(HALF, HALF)], sem_b),
        pltpu.async_copy(m2_hbm.at[pl.ds(base + HALF, HALF)],
                         m2_v.at[pl.ds(HALF, HALF)], sem_b),
    ]
    for c in in_a:
        c.wait()

    par = par_v[...]
    t0 = jnp.full((LANES,), par[0], jnp.float32)
    t1 = jnp.full((LANES,), par[1], jnp.float32)
    nw0 = jnp.full((LANES,), par[2], jnp.float32)
    nw1 = jnp.full((LANES,), par[3], jnp.float32)
    nbb = jnp.full((LANES,), par[4], jnp.float32)

    def step(i, _):
        for j in range(UNROLL):
            off = i * (UNROLL * LANES) + j * LANES
            ids = m1_v[pl.ds(off, LANES)]
            m2c = m2_v[pl.ds(off, LANES)]
            odd = (ids & jnp.int32(1)) == jnp.int32(1)
            emb = jnp.where(odd, t1, t0)
            zn = emb * nw0 + m2c * nw1 + nbb
            out_v[pl.ds(off, LANES)] = 1.0 / (1.0 + jnp.exp(zn))
        return _

    half_iters = (STEPS // 2) // UNROLL
    lax.fori_loop(0, half_iters, step, 0)
    out_a = pltpu.async_copy(out_v.at[pl.ds(0, HALF)],
                             out_hbm.at[pl.ds(base, HALF)], sem_o)
    for c in in_b:
        c.wait()
    lax.fori_loop(half_iters, 2 * half_iters, step, 0)
    out_b = pltpu.async_copy(out_v.at[pl.ds(HALF, HALF)],
                             out_hbm.at[pl.ds(base + HALF, HALF)], sem_o)
    out_a.wait()
    out_b.wait()


def kernel(m1, m2, emb_table, W, b):
    m1_flat = m1.reshape(-1).astype(jnp.int32)
    m2_flat = m2.reshape(-1).astype(jnp.float32)
    tab = emb_table.reshape(-1).astype(jnp.float32)
    w = W.reshape(-1).astype(jnp.float32)
    scal = jnp.stack([tab[0], tab[1], -w[0], -w[1], -b.reshape(-1)[0]])
    params = jnp.concatenate([scal, jnp.zeros((LANES - 5,), jnp.float32)])
    out = _sc_forward(m1_flat, m2_flat, params)
    return out.reshape(BATCH, 1)


# speedup vs baseline: 1.0061x; 1.0061x over previous
import functools

import jax
import jax.numpy as jnp
from jax import lax
from jax.experimental import pallas as pl
from jax.experimental.pallas import tpu as pltpu
from jax.experimental.pallas import tpu_sc as plsc

BATCH = 16384
NUM_WORKERS = 32
CHUNK = BATCH // NUM_WORKERS
LANES = 16
STEPS = CHUNK // LANES
UNROLL = 4


@functools.partial(
    pl.kernel,
    mesh=plsc.VectorSubcoreMesh(core_axis_name="c", subcore_axis_name="s"),
    out_type=jax.ShapeDtypeStruct((BATCH,), jnp.float32),
    scratch_types=[
        pltpu.VMEM((CHUNK,), jnp.int32),
        pltpu.VMEM((CHUNK,), jnp.float32),
        pltpu.VMEM((CHUNK,), jnp.float32),
        pltpu.VMEM((LANES,), jnp.float32),
        pltpu.SemaphoreType.DMA,
    ],
)
def _sc_forward(m1_hbm, m2_hbm, par_hbm, out_hbm,
                m1_v, m2_v, out_v, par_v, sem):
    wid = lax.axis_index("s") * 2 + lax.axis_index("c")
    base = wid * CHUNK
    copies = [
        pltpu.async_copy(m1_hbm.at[pl.ds(base, CHUNK)], m1_v, sem),
        pltpu.async_copy(m2_hbm.at[pl.ds(base, CHUNK)], m2_v, sem),
        pltpu.async_copy(par_hbm, par_v, sem),
    ]
    for c in copies:
        c.wait()

    par = par_v[...]
    t0 = jnp.full((LANES,), par[0], jnp.float32)
    t1 = jnp.full((LANES,), par[1], jnp.float32)
    nw0 = jnp.full((LANES,), par[2], jnp.float32)
    nw1 = jnp.full((LANES,), par[3], jnp.float32)
    nbb = jnp.full((LANES,), par[4], jnp.float32)

    def step(i, _):
        off = i * LANES
        ids = m1_v[pl.ds(off, LANES)]
        m2c = m2_v[pl.ds(off, LANES)]
        odd = (ids & jnp.int32(1)) == jnp.int32(1)
        emb = jnp.where(odd, t1, t0)
        zn = emb * nw0 + m2c * nw1 + nbb
        out_v[pl.ds(off, LANES)] = 1.0 / (1.0 + jnp.exp(zn))
        return _

    lax.fori_loop(0, STEPS, step, 0)
    pltpu.sync_copy(out_v, out_hbm.at[pl.ds(base, CHUNK)])


def kernel(m1, m2, emb_table, W, b):
    m1_flat = m1.reshape(-1).astype(jnp.int32)
    m2_flat = m2.reshape(-1).astype(jnp.float32)
    tab = emb_table.reshape(-1).astype(jnp.float32)
    w = W.reshape(-1).astype(jnp.float32)
    scal = jnp.stack([tab[0], tab[1], -w[0], -w[1], -b.reshape(-1)[0]])
    params = jnp.concatenate([scal, jnp.zeros((LANES - 5,), jnp.float32)])
    out = _sc_forward(m1_flat, m2_flat, params)
    return out.reshape(BATCH, 1)
